# baseline (speedup 1.0000x reference)
import functools

import jax
import jax.numpy as jnp
from jax import lax
from jax.experimental import pallas as pl
from jax.experimental.pallas import tpu as pltpu
from jax.experimental.pallas import tpu_sc as plsc

N = 10000
E = 320000
NUM_HE = 10000
G = 16
NHID = 128
NMACRO = 512
NEG_SLOPE = 0.1

NC = 2
NS = 16
NW = NC * NS
PPW = E // NW
CHUNK = 128
NFULL = PPW // CHUNK
TAIL = PPW - NFULL * CHUNK
RPT = (N // NS) // 8 * 8
EXTRA = N - NS * RPT

def _lrelu(v):
    return jnp.where(v > 0, v, NEG_SLOPE * v)


NCHK1 = -(-PPW // CHUNK)
PAD1 = NCHK1 * CHUNK - PPW
SENT1 = N
RPTW = 640
ACC1 = NS * RPTW
SCH = 112


def _sc_counts_body(nd_idx, he_idx, pin4, ones_h, mc_idx, z1,
                    f0_o0, f1_o0, f2_o0, f3_o0, ct_o0,
                    f0_o1, f1_o1, f2_o1, f3_o1, ct_o1,
                    nd_o0, nd_o1, mc_o0, mc_o1,
                    ndb, heb, pfb, mb, onesb,
                    af0, af1, af2, af3, act, acc_nd, acc_mc):
    c = lax.axis_index("c")
    s = lax.axis_index("s")
    wid = s * NC + c

    pltpu.sync_copy(nd_idx.at[wid], ndb)
    pltpu.sync_copy(he_idx.at[wid], heb)
    pltpu.sync_copy(pin4.at[wid], pfb)
    pltpu.sync_copy(ones_h, onesb)
    pltpu.sync_copy(mc_idx.at[pl.ds(wid * 16, 16)], mb.at[0])

    sl = pl.ds(s * RPTW, RPTW)
    for a in (af0, af1, af2, af3, act, acc_nd, acc_mc):
        pltpu.sync_copy(z1, a.at[sl])

    plsc.subcore_barrier()

    @pl.loop(0, NCHK1)
    def _pins(i):
        pltpu.sync_copy(pfb.at[0, i], af0.at[heb.at[i]], add=True)
        pltpu.sync_copy(pfb.at[1, i], af1.at[heb.at[i]], add=True)
        pltpu.sync_copy(pfb.at[2, i], af2.at[heb.at[i]], add=True)
        pltpu.sync_copy(pfb.at[3, i], af3.at[heb.at[i]], add=True)
        pltpu.sync_copy(onesb, act.at[heb.at[i]], add=True)
        pltpu.sync_copy(onesb, acc_nd.at[ndb.at[i]], add=True)

    pltpu.sync_copy(onesb.at[pl.ds(0, 16)], acc_mc.at[mb.at[0]], add=True)

    plsc.subcore_barrier()

    @pl.when(c == 0)
    def _w0():
        for a, o in ((af0, f0_o0), (af1, f1_o0), (af2, f2_o0), (af3, f3_o0),
                     (act, ct_o0), (acc_nd, nd_o0), (acc_mc, mc_o0)):
            pltpu.sync_copy(a.at[sl], o.at[sl])

    @pl.when(c == 1)
    def _w1():
        for a, o in ((af0, f0_o1), (af1, f1_o1), (af2, f2_o1), (af3, f3_o1),
                     (act, ct_o1), (acc_nd, nd_o1), (acc_mc, mc_o1)):
            pltpu.sync_copy(a.at[sl], o.at[sl])


def _sc_spmm_body(table, gidx, sidx, zinit, out,
                  gb, sb0, sb1, rows0, rows1, acc, sem0, sem1, semS0, semS1):
    c = lax.axis_index("c")
    s = lax.axis_index("s")
    wid = s * NC + c
    pltpu.sync_copy(gidx.at[wid], gb)
    pltpu.sync_copy(zinit, acc.at[pl.ds(s * RPT, RPT)])

    @pl.when(s == 0)
    def _init_extra():
        pltpu.sync_copy(zinit.at[pl.ds(0, EXTRA)], acc.at[pl.ds(NS * RPT, EXTRA)])

    plsc.subcore_barrier()

    pltpu.async_copy(table.at[gb.at[0]], rows0, sem0)
    pltpu.sync_copy(sidx.at[wid, 0], sb0.at[0])
    pltpu.async_copy(table.at[gb.at[1]], rows1, sem1)
    pltpu.sync_copy(sidx.at[wid, 1], sb1.at[0])

    @pl.loop(0, NCHK1 // 2)
    def _pairs(p):
        i0 = 2 * p
        pltpu.make_async_copy(table.at[gb.at[i0]], rows0, sem0).wait()
        pltpu.async_copy(rows0, acc.at[sb0.at[0]], semS0, add=True)
        pltpu.make_async_copy(table.at[gb.at[i0 + 1]], rows1, sem1).wait()
        pltpu.async_copy(rows1, acc.at[sb1.at[0]], semS1, add=True)

        @pl.when(i0 + 2 < NCHK1)
        def _refill0():
            pltpu.make_async_copy(rows0, acc.at[sb0.at[0]], semS0).wait()
            pltpu.async_copy(table.at[gb.at[i0 + 2]], rows0, sem0)
            pltpu.sync_copy(sidx.at[wid, i0 + 2], sb0.at[0])

        @pl.when(i0 + 3 < NCHK1)
        def _refill1():
            pltpu.make_async_copy(rows1, acc.at[sb1.at[0]], semS1).wait()
            pltpu.async_copy(table.at[gb.at[i0 + 3]], rows1, sem1)
            pltpu.sync_copy(sidx.at[wid, i0 + 3], sb1.at[0])

    pltpu.make_async_copy(table.at[gb.at[NCHK1 - 1]], rows0, sem0).wait()
    pltpu.sync_copy(rows0, acc.at[sb0.at[0]], add=True)
    pltpu.make_async_copy(rows1, acc.at[sb1.at[0]], semS1).wait()

    plsc.subcore_barrier()
    sl = pl.ds(s * RPT, RPT)
    pltpu.sync_copy(acc.at[sl], out.at[c, sl])

    @pl.when(s == 0)
    def _out_extra():
        ex = pl.ds(NS * RPT, EXTRA)
        pltpu.sync_copy(acc.at[ex], out.at[c, ex])


@functools.cache
def _sc_kernels():
    mesh = plsc.VectorSubcoreMesh(core_axis_name="c", subcore_axis_name="s",
                                  num_cores=NC, num_subcores=NS)
    sc_counts = pl.kernel(
        _sc_counts_body,
        out_type=tuple(
            jax.ShapeDtypeStruct((ACC1,), jnp.float32) for _ in range(14)),
        mesh=mesh,
        scratch_types=[
            pltpu.VMEM((NCHK1, CHUNK), jnp.int32),
            pltpu.VMEM((NCHK1, CHUNK), jnp.int32),
            pltpu.VMEM((4, NCHK1, CHUNK), jnp.float32),
            pltpu.VMEM((1, 16), jnp.int32),
            pltpu.VMEM((CHUNK,), jnp.float32),
            pltpu.VMEM_SHARED((ACC1,), jnp.float32),
            pltpu.VMEM_SHARED((ACC1,), jnp.float32),
            pltpu.VMEM_SHARED((ACC1,), jnp.float32),
            pltpu.VMEM_SHARED((ACC1,), jnp.float32),
            pltpu.VMEM_SHARED((ACC1,), jnp.float32),
            pltpu.VMEM_SHARED((ACC1,), jnp.float32),
            pltpu.VMEM_SHARED((ACC1,), jnp.float32),
        ],
    )
    sc_spmm = pl.kernel(
        _sc_spmm_body,
        out_type=jax.ShapeDtypeStruct((NC, N, NHID), jnp.float32),
        mesh=mesh,
        scratch_types=[
            pltpu.VMEM((NCHK1, CHUNK), jnp.int32),
            pltpu.VMEM((1, CHUNK), jnp.int32),
            pltpu.VMEM((1, CHUNK), jnp.int32),
            pltpu.VMEM((CHUNK, NHID), jnp.float32),
            pltpu.VMEM((CHUNK, NHID), jnp.float32),
            pltpu.VMEM_SHARED((N + 8, NHID), jnp.float32),
            pltpu.SemaphoreType.DMA,
            pltpu.SemaphoreType.DMA,
            pltpu.SemaphoreType.DMA,
            pltpu.SemaphoreType.DMA,
        ],
    )
    return sc_counts, sc_spmm


BLK = 1000


def _col4(v):
    m = lax.broadcasted_iota(jnp.int32, v.shape, 1) == 4
    return jnp.sum(jnp.where(m, v, 0.0), axis=-1, keepdims=True)


def _tc_xw_body(xin_ref, w_ref, wrow_ref, mc_ref, out_ref):
    ism = jnp.minimum(mc_ref[0] + mc_ref[1], 1.0)
    out_ref[...] = (
        jnp.dot(xin_ref[...], w_ref[...], preferred_element_type=jnp.float32)
        + ism * wrow_ref[...]
    )


def _tc_me_body(ma_ref, php_ref, wp5_ref, out_ref):
    php = php_ref[0] + php_ref[1]
    cnt = jnp.maximum(_col4(php), 1.0)
    pin = jnp.dot(php, wp5_ref[...], preferred_element_type=jnp.float32)
    out_ref[...] = (ma_ref[0] + ma_ref[1] + pin) / cnt


def _tc_pool_body(ob_ref, ncp_ref, mcp_ref, b1_ref, batch_ref,
                  asum_ref, acnt_ref, msum_ref, mcnt_ref):
    ncnt = jnp.maximum(ncp_ref[0] + ncp_ref[1], 1.0)
    h = _lrelu((ob_ref[0] + ob_ref[1]) / ncnt + b1_ref[...])
    mw = mcp_ref[0] + mcp_ref[1]
    onehot = (batch_ref[...]
              == lax.broadcasted_iota(jnp.int32, (BLK, G), 1)).astype(jnp.float32)
    woh = onehot * mw
    ones = jnp.ones_like(h)
    dn = (((0,), (0,)), ((), ()))
    c_as = lax.dot_general(onehot, h, dn, preferred_element_type=jnp.float32)
    c_ac = lax.dot_general(onehot, ones, dn, preferred_element_type=jnp.float32)
    c_ms = lax.dot_general(woh, h, dn, preferred_element_type=jnp.float32)
    c_mc = lax.dot_general(woh, ones, dn, preferred_element_type=jnp.float32)

    @pl.when(pl.program_id(0) == 0)
    def _():
        asum_ref[...] = jnp.zeros_like(asum_ref)
        acnt_ref[...] = jnp.zeros_like(acnt_ref)
        msum_ref[...] = jnp.zeros_like(msum_ref)
        mcnt_ref[...] = jnp.zeros_like(mcnt_ref)

    asum_ref[...] += c_as
    acnt_ref[...] += c_ac
    msum_ref[...] += c_ms
    mcnt_ref[...] += c_mc


def _tc_head_body(asum_ref, acnt_ref, msum_ref, mcnt_ref, meta_ref,
                  wm_ref, bm_ref, m1w_ref, m1b_ref, m2w_ref, m2b_ref,
                  m3w_ref, m3b_ref, out_ref):
    gap_all = asum_ref[...] / jnp.maximum(acnt_ref[...], 1.0)
    gap_mac = msum_ref[...] / jnp.maximum(mcnt_ref[...], 1.0)
    meta = _lrelu(
        jnp.dot(meta_ref[...], wm_ref[...], preferred_element_type=jnp.float32)
        + bm_ref[...])
    z = jnp.concatenate([gap_mac, gap_all, meta], axis=1)
    z = _lrelu(jnp.dot(z, m1w_ref[...], preferred_element_type=jnp.float32)
               + m1b_ref[...])
    z = _lrelu(jnp.dot(z, m2w_ref[...], preferred_element_type=jnp.float32)
               + m2b_ref[...])
    out_ref[...] = (jnp.dot(z, m3w_ref[...], preferred_element_type=jnp.float32)
                    + m3b_ref[...])


def _full(shape):
    return pl.BlockSpec(shape, lambda: tuple(0 for _ in shape))


_tc_xw = pl.pallas_call(
    _tc_xw_body,
    grid=(N // BLK,),
    in_specs=[
        pl.BlockSpec((BLK, NHID), lambda i: (i, 0)),
        pl.BlockSpec((NHID, NHID), lambda i: (0, 0)),
        pl.BlockSpec((1, NHID), lambda i: (0, 0)),
        pl.BlockSpec((NC, BLK, 1), lambda i: (0, i, 0)),
    ],
    out_specs=pl.BlockSpec((BLK, NHID), lambda i: (i, 0)),
    out_shape=jax.ShapeDtypeStruct((N, NHID), jnp.float32),
)

_tc_me = pl.pallas_call(
    _tc_me_body,
    grid=(N // BLK,),
    in_specs=[
        pl.BlockSpec((NC, BLK, NHID), lambda i: (0, i, 0)),
        pl.BlockSpec((NC, BLK, 5), lambda i: (0, i, 0)),
        pl.BlockSpec((5, NHID), lambda i: (0, 0)),
    ],
    out_specs=pl.BlockSpec((BLK, NHID), lambda i: (i, 0)),
    out_shape=jax.ShapeDtypeStruct((N, NHID), jnp.float32),
)

_tc_pool = pl.pallas_call(
    _tc_pool_body,
    grid=(N // BLK,),
    in_specs=[
        pl.BlockSpec((NC, BLK, NHID), lambda i: (0, i, 0)),
        pl.BlockSpec((NC, BLK, 1), lambda i: (0, i, 0)),
        pl.BlockSpec((NC, BLK, 1), lambda i: (0, i, 0)),
        pl.BlockSpec((1, NHID), lambda i: (0, 0)),
        pl.BlockSpec((BLK, 1), lambda i: (i, 0)),
    ],
    out_specs=[
        pl.BlockSpec((G, NHID), lambda i: (0, 0)),
        pl.BlockSpec((G, NHID), lambda i: (0, 0)),
        pl.BlockSpec((G, NHID), lambda i: (0, 0)),
        pl.BlockSpec((G, NHID), lambda i: (0, 0)),
    ],
    out_shape=[
        jax.ShapeDtypeStruct((G, NHID), jnp.float32),
        jax.ShapeDtypeStruct((G, NHID), jnp.float32),
        jax.ShapeDtypeStruct((G, NHID), jnp.float32),
        jax.ShapeDtypeStruct((G, NHID), jnp.float32),
    ],
)

_tc_head = pl.pallas_call(
    _tc_head_body,
    in_specs=[
        _full((G, NHID)), _full((G, NHID)), _full((G, NHID)), _full((G, NHID)),
        _full((G, 13)),
        _full((13, NHID)), _full((1, NHID)),
        _full((3 * NHID, NHID)), _full((1, NHID)),
        _full((NHID, NHID // 2)), _full((1, NHID // 2)),
        _full((NHID // 2, 4)), _full((1, 4)),
    ],
    out_specs=_full((G, 4)),
    out_shape=jax.ShapeDtypeStruct((G, 4), jnp.float32),
)


def kernel(x, edge_index, pin_feature, fake_pos, batch, macro_index, meta_feature,
           W1, Wp, b1, Wm, bm, M1W, M1b, M2W, M2b, M3W, M3b):
    f32 = jnp.float32
    ei = edge_index.astype(jnp.int32)
    node = ei[0]
    he = ei[1]
    batch2 = batch.astype(jnp.int32).reshape(N, 1)
    macro32 = macro_index.astype(jnp.int32)

    i32 = jnp.int32

    def _pad_chunks(v, padval):
        return jnp.concatenate(
            [v.reshape(NW, PPW), jnp.full((NW, PAD1), padval, i32)], axis=1
        ).reshape(NW, NCHK1, CHUNK)

    node_g = _pad_chunks(node, 0)
    node_s = _pad_chunks(node, N)
    he_g = _pad_chunks(he, 0)
    he_s = _pad_chunks(he, N)

    pin4 = jnp.concatenate(
        [pin_feature.T.reshape(4, NW, PPW), jnp.zeros((4, NW, PAD1), f32)],
        axis=2).reshape(4, NW, NCHK1, CHUNK).transpose(1, 0, 2, 3)

    ones128 = jnp.ones((CHUNK,), f32)
    z1 = jnp.zeros((RPTW,), f32)
    z128 = jnp.zeros((RPT, NHID), f32)

    sc_counts, sc_spmm = _sc_kernels()
    (f00, f10, f20, f30, ct0, f01, f11, f21, f31, ct1,
     nd0, nd1, mc0, mc1) = sc_counts(node_s, he_s, pin4, ones128, macro32, z1)
    php = jnp.stack([
        jnp.stack([f00[:N], f10[:N], f20[:N], f30[:N], ct0[:N]], axis=-1),
        jnp.stack([f01[:N], f11[:N], f21[:N], f31[:N], ct1[:N]], axis=-1)])
    ncp = jnp.stack([nd0[:N], nd1[:N]])[:, :, None]
    mcp = jnp.stack([mc0[:N], mc1[:N]])[:, :, None]

    xin = jnp.concatenate([x, fake_pos, jnp.zeros((N, 1), f32)], axis=1)
    xw = _tc_xw(xin, W1, W1[NHID - 1:NHID, :], mcp)

    me_part = sc_spmm(xw, node_g, he_s, z128)

    wp5 = jnp.concatenate([Wp, jnp.zeros((1, NHID), f32)], axis=0)
    m_e = _tc_me(me_part, php, wp5)

    out_part = sc_spmm(m_e, he_g, node_s, z128)

    asum, acnt, msum, mcnt = _tc_pool(
        out_part, ncp, mcp, b1.reshape(1, NHID), batch2)

    return _tc_head(asum, acnt, msum, mcnt, meta_feature,
                    Wm, bm.reshape(1, NHID), M1W, M1b.reshape(1, NHID),
                    M2W, M2b.reshape(1, NHID // 2), M3W, M3b.reshape(1, 4))


# speedup vs baseline: 1.0208x; 1.0208x over previous
import functools

import jax
import jax.numpy as jnp
from jax import lax
from jax.experimental import pallas as pl
from jax.experimental.pallas import tpu as pltpu
from jax.experimental.pallas import tpu_sc as plsc

N = 10000
E = 320000
NUM_HE = 10000
G = 16
NHID = 128
NMACRO = 512
NEG_SLOPE = 0.1

NC = 2
NS = 16
NW = NC * NS
PPW = E // NW
CHUNK = 128
NFULL = PPW // CHUNK
TAIL = PPW - NFULL * CHUNK
RPT = (N // NS) // 8 * 8
EXTRA = N - NS * RPT

def _lrelu(v):
    return jnp.where(v > 0, v, NEG_SLOPE * v)


NCHK1 = -(-PPW // CHUNK)
PAD1 = NCHK1 * CHUNK - PPW
SENT1 = N
RPTW = 640
ACC1 = NS * RPTW
SCH = 112


def _sc_counts_body(nd_idx, he_idx, pin4, ones_h, mc_idx, z1,
                    f0_o0, f1_o0, f2_o0, f3_o0, ct_o0,
                    f0_o1, f1_o1, f2_o1, f3_o1, ct_o1,
                    nd_o0, nd_o1, mc_o0, mc_o1,
                    ndb, heb, pfb, mb, onesb,
                    af0, af1, af2, af3, act, acc_nd, acc_mc):
    c = lax.axis_index("c")
    s = lax.axis_index("s")
    wid = s * NC + c

    pltpu.sync_copy(nd_idx.at[wid], ndb)
    pltpu.sync_copy(he_idx.at[wid], heb)
    pltpu.sync_copy(pin4.at[wid], pfb)
    pltpu.sync_copy(ones_h, onesb)
    pltpu.sync_copy(mc_idx.at[pl.ds(wid * 16, 16)], mb.at[0])

    sl = pl.ds(s * RPTW, RPTW)
    for a in (af0, af1, af2, af3, act, acc_nd, acc_mc):
        pltpu.sync_copy(z1, a.at[sl])

    plsc.subcore_barrier()

    @pl.loop(0, NCHK1)
    def _pins(i):
        pltpu.sync_copy(pfb.at[0, i], af0.at[heb.at[i]], add=True)
        pltpu.sync_copy(pfb.at[1, i], af1.at[heb.at[i]], add=True)
        pltpu.sync_copy(pfb.at[2, i], af2.at[heb.at[i]], add=True)
        pltpu.sync_copy(pfb.at[3, i], af3.at[heb.at[i]], add=True)
        pltpu.sync_copy(onesb, act.at[heb.at[i]], add=True)
        pltpu.sync_copy(onesb, acc_nd.at[ndb.at[i]], add=True)

    pltpu.sync_copy(onesb.at[pl.ds(0, 16)], acc_mc.at[mb.at[0]], add=True)

    plsc.subcore_barrier()

    @pl.when(c == 0)
    def _w0():
        for a, o in ((af0, f0_o0), (af1, f1_o0), (af2, f2_o0), (af3, f3_o0),
                     (act, ct_o0), (acc_nd, nd_o0), (acc_mc, mc_o0)):
            pltpu.sync_copy(a.at[sl], o.at[sl])

    @pl.when(c == 1)
    def _w1():
        for a, o in ((af0, f0_o1), (af1, f1_o1), (af2, f2_o1), (af3, f3_o1),
                     (act, ct_o1), (acc_nd, nd_o1), (acc_mc, mc_o1)):
            pltpu.sync_copy(a.at[sl], o.at[sl])


def _sc_spmm_body(table, gidx, sidx, zinit, out,
                  gb, sb0, sb1, rows0, rows1, acc, sem0, sem1):
    c = lax.axis_index("c")
    s = lax.axis_index("s")
    wid = s * NC + c
    pltpu.sync_copy(gidx.at[wid], gb)
    pltpu.sync_copy(zinit, acc.at[pl.ds(s * RPT, RPT)])

    @pl.when(s == 0)
    def _init_extra():
        pltpu.sync_copy(zinit.at[pl.ds(0, EXTRA)], acc.at[pl.ds(NS * RPT, EXTRA)])

    plsc.subcore_barrier()

    pltpu.async_copy(table.at[gb.at[0]], rows0, sem0)
    pltpu.sync_copy(sidx.at[wid, 0], sb0.at[0])

    @pl.loop(0, NCHK1 // 2)
    def _pairs(p):
        i0 = 2 * p
        pltpu.async_copy(table.at[gb.at[i0 + 1]], rows1, sem1)
        pltpu.sync_copy(sidx.at[wid, i0 + 1], sb1.at[0])
        pltpu.make_async_copy(table.at[gb.at[i0]], rows0, sem0).wait()
        pltpu.sync_copy(rows0, acc.at[sb0.at[0]], add=True)

        @pl.when(i0 + 2 < NCHK1)
        def _next():
            pltpu.async_copy(table.at[gb.at[i0 + 2]], rows0, sem0)
            pltpu.sync_copy(sidx.at[wid, i0 + 2], sb0.at[0])

        pltpu.make_async_copy(table.at[gb.at[i0 + 1]], rows1, sem1).wait()
        pltpu.sync_copy(rows1, acc.at[sb1.at[0]], add=True)

    pltpu.make_async_copy(table.at[gb.at[NCHK1 - 1]], rows0, sem0).wait()
    pltpu.sync_copy(rows0, acc.at[sb0.at[0]], add=True)

    plsc.subcore_barrier()
    sl = pl.ds(s * RPT, RPT)
    pltpu.sync_copy(acc.at[sl], out.at[c, sl])

    @pl.when(s == 0)
    def _out_extra():
        ex = pl.ds(NS * RPT, EXTRA)
        pltpu.sync_copy(acc.at[ex], out.at[c, ex])


@functools.cache
def _sc_kernels():
    mesh = plsc.VectorSubcoreMesh(core_axis_name="c", subcore_axis_name="s",
                                  num_cores=NC, num_subcores=NS)
    sc_counts = pl.kernel(
        _sc_counts_body,
        out_type=tuple(
            jax.ShapeDtypeStruct((ACC1,), jnp.float32) for _ in range(14)),
        mesh=mesh,
        scratch_types=[
            pltpu.VMEM((NCHK1, CHUNK), jnp.int32),
            pltpu.VMEM((NCHK1, CHUNK), jnp.int32),
            pltpu.VMEM((4, NCHK1, CHUNK), jnp.float32),
            pltpu.VMEM((1, 16), jnp.int32),
            pltpu.VMEM((CHUNK,), jnp.float32),
            pltpu.VMEM_SHARED((ACC1,), jnp.float32),
            pltpu.VMEM_SHARED((ACC1,), jnp.float32),
            pltpu.VMEM_SHARED((ACC1,), jnp.float32),
            pltpu.VMEM_SHARED((ACC1,), jnp.float32),
            pltpu.VMEM_SHARED((ACC1,), jnp.float32),
            pltpu.VMEM_SHARED((ACC1,), jnp.float32),
            pltpu.VMEM_SHARED((ACC1,), jnp.float32),
        ],
    )
    sc_spmm = pl.kernel(
        _sc_spmm_body,
        out_type=jax.ShapeDtypeStruct((NC, N, NHID), jnp.float32),
        mesh=mesh,
        scratch_types=[
            pltpu.VMEM((NCHK1, CHUNK), jnp.int32),
            pltpu.VMEM((1, CHUNK), jnp.int32),
            pltpu.VMEM((1, CHUNK), jnp.int32),
            pltpu.VMEM((CHUNK, NHID), jnp.float32),
            pltpu.VMEM((CHUNK, NHID), jnp.float32),
            pltpu.VMEM_SHARED((N + 8, NHID), jnp.float32),
            pltpu.SemaphoreType.DMA,
            pltpu.SemaphoreType.DMA,
        ],
    )
    return sc_counts, sc_spmm


BLK = 1000


def _col4(v):
    m = lax.broadcasted_iota(jnp.int32, v.shape, 1) == 4
    return jnp.sum(jnp.where(m, v, 0.0), axis=-1, keepdims=True)


def _tc_xw_body(xin_ref, w_ref, wrow_ref, mc_ref, out_ref):
    ism = jnp.minimum(mc_ref[0] + mc_ref[1], 1.0)
    out_ref[...] = (
        jnp.dot(xin_ref[...], w_ref[...], preferred_element_type=jnp.float32)
        + ism * wrow_ref[...]
    )


def _tc_me_body(ma_ref, php_ref, wp5_ref, out_ref):
    php = php_ref[0] + php_ref[1]
    cnt = jnp.maximum(_col4(php), 1.0)
    pin = jnp.dot(php, wp5_ref[...], preferred_element_type=jnp.float32)
    out_ref[...] = (ma_ref[0] + ma_ref[1] + pin) / cnt


def _tc_poolhead_body(ob_ref, ncp_ref, mcp_ref, b1_ref, batch_ref, meta_ref,
                      wm_ref, bm_ref, m1w_ref, m1b_ref, m2w_ref, m2b_ref,
                      m3w_ref, m3b_ref, out_ref,
                      asum_ref, acnt_ref, msum_ref, mcnt_ref):
    ncnt = jnp.maximum(ncp_ref[0] + ncp_ref[1], 1.0)
    h = _lrelu((ob_ref[0] + ob_ref[1]) / ncnt + b1_ref[...])
    mw = mcp_ref[0] + mcp_ref[1]
    onehot = (batch_ref[...]
              == lax.broadcasted_iota(jnp.int32, (BLK, G), 1)).astype(jnp.float32)
    woh = onehot * mw
    ones = jnp.ones_like(h)
    dn = (((0,), (0,)), ((), ()))
    c_as = lax.dot_general(onehot, h, dn, preferred_element_type=jnp.float32)
    c_ac = lax.dot_general(onehot, ones, dn, preferred_element_type=jnp.float32)
    c_ms = lax.dot_general(woh, h, dn, preferred_element_type=jnp.float32)
    c_mc = lax.dot_general(woh, ones, dn, preferred_element_type=jnp.float32)

    @pl.when(pl.program_id(0) == 0)
    def _():
        asum_ref[...] = jnp.zeros_like(asum_ref)
        acnt_ref[...] = jnp.zeros_like(acnt_ref)
        msum_ref[...] = jnp.zeros_like(msum_ref)
        mcnt_ref[...] = jnp.zeros_like(mcnt_ref)

    asum_ref[...] += c_as
    acnt_ref[...] += c_ac
    msum_ref[...] += c_ms
    mcnt_ref[...] += c_mc

    @pl.when(pl.program_id(0) == N // BLK - 1)
    def _head():
        gap_all = asum_ref[...] / jnp.maximum(acnt_ref[...], 1.0)
        gap_mac = msum_ref[...] / jnp.maximum(mcnt_ref[...], 1.0)
        meta = _lrelu(
            jnp.dot(meta_ref[...], wm_ref[...], preferred_element_type=jnp.float32)
            + bm_ref[...])
        z = jnp.concatenate([gap_mac, gap_all, meta], axis=1)
        z = _lrelu(jnp.dot(z, m1w_ref[...], preferred_element_type=jnp.float32)
                   + m1b_ref[...])
        z = _lrelu(jnp.dot(z, m2w_ref[...], preferred_element_type=jnp.float32)
                   + m2b_ref[...])
        out_ref[...] = (jnp.dot(z, m3w_ref[...],
                                preferred_element_type=jnp.float32)
                        + m3b_ref[...])


def _full(shape):
    return pl.BlockSpec(shape, lambda: tuple(0 for _ in shape))


_tc_xw = pl.pallas_call(
    _tc_xw_body,
    grid=(N // BLK,),
    in_specs=[
        pl.BlockSpec((BLK, NHID), lambda i: (i, 0)),
        pl.BlockSpec((NHID, NHID), lambda i: (0, 0)),
        pl.BlockSpec((1, NHID), lambda i: (0, 0)),
        pl.BlockSpec((NC, BLK, 1), lambda i: (0, i, 0)),
    ],
    out_specs=pl.BlockSpec((BLK, NHID), lambda i: (i, 0)),
    out_shape=jax.ShapeDtypeStruct((N, NHID), jnp.float32),
)

_tc_me = pl.pallas_call(
    _tc_me_body,
    grid=(N // BLK,),
    in_specs=[
        pl.BlockSpec((NC, BLK, NHID), lambda i: (0, i, 0)),
        pl.BlockSpec((NC, BLK, 5), lambda i: (0, i, 0)),
        pl.BlockSpec((5, NHID), lambda i: (0, 0)),
    ],
    out_specs=pl.BlockSpec((BLK, NHID), lambda i: (i, 0)),
    out_shape=jax.ShapeDtypeStruct((N, NHID), jnp.float32),
)

_tc_poolhead = pl.pallas_call(
    _tc_poolhead_body,
    grid=(N // BLK,),
    in_specs=[
        pl.BlockSpec((NC, BLK, NHID), lambda i: (0, i, 0)),
        pl.BlockSpec((NC, BLK, 1), lambda i: (0, i, 0)),
        pl.BlockSpec((NC, BLK, 1), lambda i: (0, i, 0)),
        pl.BlockSpec((1, NHID), lambda i: (0, 0)),
        pl.BlockSpec((BLK, 1), lambda i: (i, 0)),
        pl.BlockSpec((G, 13), lambda i: (0, 0)),
        pl.BlockSpec((13, NHID), lambda i: (0, 0)),
        pl.BlockSpec((1, NHID), lambda i: (0, 0)),
        pl.BlockSpec((3 * NHID, NHID), lambda i: (0, 0)),
        pl.BlockSpec((1, NHID), lambda i: (0, 0)),
        pl.BlockSpec((NHID, NHID // 2), lambda i: (0, 0)),
        pl.BlockSpec((1, NHID // 2), lambda i: (0, 0)),
        pl.BlockSpec((NHID // 2, 4), lambda i: (0, 0)),
        pl.BlockSpec((1, 4), lambda i: (0, 0)),
    ],
    out_specs=pl.BlockSpec((G, 4), lambda i: (0, 0)),
    out_shape=jax.ShapeDtypeStruct((G, 4), jnp.float32),
    scratch_shapes=[
        pltpu.VMEM((G, NHID), jnp.float32),
        pltpu.VMEM((G, NHID), jnp.float32),
        pltpu.VMEM((G, NHID), jnp.float32),
        pltpu.VMEM((G, NHID), jnp.float32),
    ],
)


def kernel(x, edge_index, pin_feature, fake_pos, batch, macro_index, meta_feature,
           W1, Wp, b1, Wm, bm, M1W, M1b, M2W, M2b, M3W, M3b):
    f32 = jnp.float32
    ei = edge_index.astype(jnp.int32)
    node = ei[0]
    he = ei[1]
    batch2 = batch.astype(jnp.int32).reshape(N, 1)
    macro32 = macro_index.astype(jnp.int32)

    i32 = jnp.int32

    def _pad_chunks(v, padval):
        return jnp.concatenate(
            [v.reshape(NW, PPW), jnp.full((NW, PAD1), padval, i32)], axis=1
        ).reshape(NW, NCHK1, CHUNK)

    node_g = _pad_chunks(node, 0)
    node_s = _pad_chunks(node, N)
    he_g = _pad_chunks(he, 0)
    he_s = _pad_chunks(he, N)

    pin4 = jnp.concatenate(
        [pin_feature.reshape(NW, PPW, 4).transpose(0, 2, 1),
         jnp.zeros((NW, 4, PAD1), f32)], axis=2).reshape(NW, 4, NCHK1, CHUNK)

    ones128 = jnp.ones((CHUNK,), f32)
    z1 = jnp.zeros((RPTW,), f32)
    z128 = jnp.zeros((RPT, NHID), f32)

    sc_counts, sc_spmm = _sc_kernels()
    (f00, f10, f20, f30, ct0, f01, f11, f21, f31, ct1,
     nd0, nd1, mc0, mc1) = sc_counts(node_s, he_s, pin4, ones128, macro32, z1)
    php = jnp.stack([
        jnp.stack([f00[:N], f10[:N], f20[:N], f30[:N], ct0[:N]], axis=-1),
        jnp.stack([f01[:N], f11[:N], f21[:N], f31[:N], ct1[:N]], axis=-1)])
    ncp = jnp.stack([nd0[:N], nd1[:N]])[:, :, None]
    mcp = jnp.stack([mc0[:N], mc1[:N]])[:, :, None]

    xin = jnp.concatenate([x, fake_pos, jnp.zeros((N, 1), f32)], axis=1)
    xw = _tc_xw(xin, W1, W1[NHID - 1:NHID, :], mcp)

    me_part = sc_spmm(xw, node_g, he_s, z128)

    wp5 = jnp.concatenate([Wp, jnp.zeros((1, NHID), f32)], axis=0)
    m_e = _tc_me(me_part, php, wp5)

    out_part = sc_spmm(m_e, he_g, node_s, z128)

    return _tc_poolhead(
        out_part, ncp, mcp, b1.reshape(1, NHID), batch2, meta_feature,
        Wm, bm.reshape(1, NHID), M1W, M1b.reshape(1, NHID),
        M2W, M2b.reshape(1, NHID // 2), M3W, M3b.reshape(1, 4))


# speedup vs baseline: 1.0210x; 1.0003x over previous
import functools

import jax
import jax.numpy as jnp
from jax import lax
from jax.experimental import pallas as pl
from jax.experimental.pallas import tpu as pltpu
from jax.experimental.pallas import tpu_sc as plsc

N = 10000
E = 320000
NUM_HE = 10000
G = 16
NHID = 128
NMACRO = 512
NEG_SLOPE = 0.1

NC = 2
NS = 16
NW = NC * NS
PPW = E // NW
CHUNK = 128
RPT = (N // NS) // 8 * 8
EXTRA = N - NS * RPT

def _lrelu(v):
    return jnp.where(v > 0, v, NEG_SLOPE * v)


NCHK1 = -(-PPW // CHUNK)
PAD1 = NCHK1 * CHUNK - PPW
SENT1 = N
RPTW = 640
ACC1 = NS * RPTW


def _sc_counts_body(nd_idx, he_idx, pin4, ones_h, mc_idx, z1,
                    f0_o0, f1_o0, f2_o0, f3_o0, ct_o0,
                    f0_o1, f1_o1, f2_o1, f3_o1, ct_o1,
                    nd_o0, nd_o1, mc_o0, mc_o1,
                    ndb, heb, pfb, mb, onesb,
                    af0, af1, af2, af3, act, acc_nd, acc_mc):
    c = lax.axis_index("c")
    s = lax.axis_index("s")
    wid = s * NC + c

    pltpu.sync_copy(nd_idx.at[wid], ndb)
    pltpu.sync_copy(he_idx.at[wid], heb)
    pltpu.sync_copy(pin4.at[wid], pfb)
    pltpu.sync_copy(ones_h, onesb)
    pltpu.sync_copy(mc_idx.at[pl.ds(wid * 16, 16)], mb.at[0])

    sl = pl.ds(s * RPTW, RPTW)
    for a in (af0, af1, af2, af3, act, acc_nd, acc_mc):
        pltpu.sync_copy(z1, a.at[sl])

    plsc.subcore_barrier()

    @pl.loop(0, NCHK1)
    def _pins(i):
        pltpu.sync_copy(pfb.at[0, i], af0.at[heb.at[i]], add=True)
        pltpu.sync_copy(pfb.at[1, i], af1.at[heb.at[i]], add=True)
        pltpu.sync_copy(pfb.at[2, i], af2.at[heb.at[i]], add=True)
        pltpu.sync_copy(pfb.at[3, i], af3.at[heb.at[i]], add=True)
        pltpu.sync_copy(onesb, act.at[heb.at[i]], add=True)
        pltpu.sync_copy(onesb, acc_nd.at[ndb.at[i]], add=True)

    pltpu.sync_copy(onesb.at[pl.ds(0, 16)], acc_mc.at[mb.at[0]], add=True)

    plsc.subcore_barrier()

    @pl.when(c == 0)
    def _w0():
        for a, o in ((af0, f0_o0), (af1, f1_o0), (af2, f2_o0), (af3, f3_o0),
                     (act, ct_o0), (acc_nd, nd_o0), (acc_mc, mc_o0)):
            pltpu.sync_copy(a.at[sl], o.at[sl])

    @pl.when(c == 1)
    def _w1():
        for a, o in ((af0, f0_o1), (af1, f1_o1), (af2, f2_o1), (af3, f3_o1),
                     (act, ct_o1), (acc_nd, nd_o1), (acc_mc, mc_o1)):
            pltpu.sync_copy(a.at[sl], o.at[sl])


def _sc_spmm_body(table, gidx, sidx, zinit, out,
                  gb, sb0, sb1, rows0, rows1, acc, sem0, sem1):
    c = lax.axis_index("c")
    s = lax.axis_index("s")
    wid = s * NC + c
    pltpu.sync_copy(gidx.at[wid], gb)
    pltpu.sync_copy(zinit, acc.at[pl.ds(s * RPT, RPT)])

    @pl.when(s == 0)
    def _init_extra():
        pltpu.sync_copy(zinit.at[pl.ds(0, EXTRA)], acc.at[pl.ds(NS * RPT, EXTRA)])

    plsc.subcore_barrier()

    pltpu.async_copy(table.at[gb.at[0]], rows0, sem0)
    pltpu.sync_copy(sidx.at[wid, 0], sb0.at[0])

    @pl.loop(0, NCHK1 // 2)
    def _pairs(p):
        i0 = 2 * p
        pltpu.async_copy(table.at[gb.at[i0 + 1]], rows1, sem1)
        pltpu.sync_copy(sidx.at[wid, i0 + 1], sb1.at[0])
        pltpu.make_async_copy(table.at[gb.at[i0]], rows0, sem0).wait()
        pltpu.sync_copy(rows0, acc.at[sb0.at[0]], add=True)

        @pl.when(i0 + 2 < NCHK1)
        def _next():
            pltpu.async_copy(table.at[gb.at[i0 + 2]], rows0, sem0)
            pltpu.sync_copy(sidx.at[wid, i0 + 2], sb0.at[0])

        pltpu.make_async_copy(table.at[gb.at[i0 + 1]], rows1, sem1).wait()
        pltpu.sync_copy(rows1, acc.at[sb1.at[0]], add=True)

    pltpu.make_async_copy(table.at[gb.at[NCHK1 - 1]], rows0, sem0).wait()
    pltpu.sync_copy(rows0, acc.at[sb0.at[0]], add=True)

    plsc.subcore_barrier()
    sl = pl.ds(s * RPT, RPT)
    pltpu.sync_copy(acc.at[sl], out.at[c, sl])

    @pl.when(s == 0)
    def _out_extra():
        ex = pl.ds(NS * RPT, EXTRA)
        pltpu.sync_copy(acc.at[ex], out.at[c, ex])


@functools.cache
def _sc_kernels():
    mesh = plsc.VectorSubcoreMesh(core_axis_name="c", subcore_axis_name="s",
                                  num_cores=NC, num_subcores=NS)
    sc_counts = pl.kernel(
        _sc_counts_body,
        out_type=tuple(
            jax.ShapeDtypeStruct((ACC1,), jnp.float32) for _ in range(14)),
        mesh=mesh,
        scratch_types=[
            pltpu.VMEM((NCHK1, CHUNK), jnp.int32),
            pltpu.VMEM((NCHK1, CHUNK), jnp.int32),
            pltpu.VMEM((4, NCHK1, CHUNK), jnp.float32),
            pltpu.VMEM((1, 16), jnp.int32),
            pltpu.VMEM((CHUNK,), jnp.float32),
            pltpu.VMEM_SHARED((ACC1,), jnp.float32),
            pltpu.VMEM_SHARED((ACC1,), jnp.float32),
            pltpu.VMEM_SHARED((ACC1,), jnp.float32),
            pltpu.VMEM_SHARED((ACC1,), jnp.float32),
            pltpu.VMEM_SHARED((ACC1,), jnp.float32),
            pltpu.VMEM_SHARED((ACC1,), jnp.float32),
            pltpu.VMEM_SHARED((ACC1,), jnp.float32),
        ],
    )
    sc_spmm = pl.kernel(
        _sc_spmm_body,
        out_type=jax.ShapeDtypeStruct((NC, N, NHID), jnp.float32),
        mesh=mesh,
        scratch_types=[
            pltpu.VMEM((NCHK1, CHUNK), jnp.int32),
            pltpu.VMEM((1, CHUNK), jnp.int32),
            pltpu.VMEM((1, CHUNK), jnp.int32),
            pltpu.VMEM((CHUNK, NHID), jnp.float32),
            pltpu.VMEM((CHUNK, NHID), jnp.float32),
            pltpu.VMEM_SHARED((N + 8, NHID), jnp.float32),
            pltpu.SemaphoreType.DMA,
            pltpu.SemaphoreType.DMA,
        ],
    )
    return sc_counts, sc_spmm


BLK = 1000


def _col4(v):
    m = lax.broadcasted_iota(jnp.int32, v.shape, 1) == 4
    return jnp.sum(jnp.where(m, v, 0.0), axis=-1, keepdims=True)


def _tc_xw_body(xin_ref, w_ref, wrow_ref, mc_ref, out_ref):
    ism = jnp.minimum(mc_ref[0] + mc_ref[1], 1.0)
    out_ref[...] = (
        jnp.dot(xin_ref[...], w_ref[...], preferred_element_type=jnp.float32)
        + ism * wrow_ref[...]
    )


def _tc_me_body(ma_ref, php_ref, wp5_ref, out_ref):
    php = php_ref[0] + php_ref[1]
    cnt = jnp.maximum(_col4(php), 1.0)
    pin = jnp.dot(php, wp5_ref[...], preferred_element_type=jnp.float32)
    out_ref[...] = (ma_ref[0] + ma_ref[1] + pin) / cnt


def _tc_poolhead_body(ob_ref, ncp_ref, mcp_ref, b1_ref, batch_ref, meta_ref,
                      wm_ref, bm_ref, m1w_ref, m1b_ref, m2w_ref, m2b_ref,
                      m3w_ref, m3b_ref, out_ref,
                      asum_ref, acnt_ref, msum_ref, mcnt_ref):
    ncnt = jnp.maximum(ncp_ref[0] + ncp_ref[1], 1.0)
    h = _lrelu((ob_ref[0] + ob_ref[1]) / ncnt + b1_ref[...])
    mw = mcp_ref[0] + mcp_ref[1]
    onehot = (batch_ref[...]
              == lax.broadcasted_iota(jnp.int32, (BLK, G), 1)).astype(jnp.float32)
    woh = onehot * mw
    ones = jnp.ones_like(h)
    dn = (((0,), (0,)), ((), ()))
    c_as = lax.dot_general(onehot, h, dn, preferred_element_type=jnp.float32)
    c_ac = lax.dot_general(onehot, ones, dn, preferred_element_type=jnp.float32)
    c_ms = lax.dot_general(woh, h, dn, preferred_element_type=jnp.float32)
    c_mc = lax.dot_general(woh, ones, dn, preferred_element_type=jnp.float32)

    @pl.when(pl.program_id(0) == 0)
    def _():
        asum_ref[...] = jnp.zeros_like(asum_ref)
        acnt_ref[...] = jnp.zeros_like(acnt_ref)
        msum_ref[...] = jnp.zeros_like(msum_ref)
        mcnt_ref[...] = jnp.zeros_like(mcnt_ref)

    asum_ref[...] += c_as
    acnt_ref[...] += c_ac
    msum_ref[...] += c_ms
    mcnt_ref[...] += c_mc

    @pl.when(pl.program_id(0) == N // BLK - 1)
    def _head():
        gap_all = asum_ref[...] / jnp.maximum(acnt_ref[...], 1.0)
        gap_mac = msum_ref[...] / jnp.maximum(mcnt_ref[...], 1.0)
        meta = _lrelu(
            jnp.dot(meta_ref[...], wm_ref[...], preferred_element_type=jnp.float32)
            + bm_ref[...])
        z = jnp.concatenate([gap_mac, gap_all, meta], axis=1)
        z = _lrelu(jnp.dot(z, m1w_ref[...], preferred_element_type=jnp.float32)
                   + m1b_ref[...])
        z = _lrelu(jnp.dot(z, m2w_ref[...], preferred_element_type=jnp.float32)
                   + m2b_ref[...])
        out_ref[...] = (jnp.dot(z, m3w_ref[...],
                                preferred_element_type=jnp.float32)
                        + m3b_ref[...])


def _full(shape):
    return pl.BlockSpec(shape, lambda: tuple(0 for _ in shape))


_tc_xw = pl.pallas_call(
    _tc_xw_body,
    grid=(N // BLK,),
    in_specs=[
        pl.BlockSpec((BLK, NHID), lambda i: (i, 0)),
        pl.BlockSpec((NHID, NHID), lambda i: (0, 0)),
        pl.BlockSpec((1, NHID), lambda i: (0, 0)),
        pl.BlockSpec((NC, BLK, 1), lambda i: (0, i, 0)),
    ],
    out_specs=pl.BlockSpec((BLK, NHID), lambda i: (i, 0)),
    out_shape=jax.ShapeDtypeStruct((N, NHID), jnp.float32),
)

_tc_me = pl.pallas_call(
    _tc_me_body,
    grid=(N // BLK,),
    in_specs=[
        pl.BlockSpec((NC, BLK, NHID), lambda i: (0, i, 0)),
        pl.BlockSpec((NC, BLK, 5), lambda i: (0, i, 0)),
        pl.BlockSpec((5, NHID), lambda i: (0, 0)),
    ],
    out_specs=pl.BlockSpec((BLK, NHID), lambda i: (i, 0)),
    out_shape=jax.ShapeDtypeStruct((N, NHID), jnp.float32),
)

_tc_poolhead = pl.pallas_call(
    _tc_poolhead_body,
    grid=(N // BLK,),
    in_specs=[
        pl.BlockSpec((NC, BLK, NHID), lambda i: (0, i, 0)),
        pl.BlockSpec((NC, BLK, 1), lambda i: (0, i, 0)),
        pl.BlockSpec((NC, BLK, 1), lambda i: (0, i, 0)),
        pl.BlockSpec((1, NHID), lambda i: (0, 0)),
        pl.BlockSpec((BLK, 1), lambda i: (i, 0)),
        pl.BlockSpec((G, 13), lambda i: (0, 0)),
        pl.BlockSpec((13, NHID), lambda i: (0, 0)),
        pl.BlockSpec((1, NHID), lambda i: (0, 0)),
        pl.BlockSpec((3 * NHID, NHID), lambda i: (0, 0)),
        pl.BlockSpec((1, NHID), lambda i: (0, 0)),
        pl.BlockSpec((NHID, NHID // 2), lambda i: (0, 0)),
        pl.BlockSpec((1, NHID // 2), lambda i: (0, 0)),
        pl.BlockSpec((NHID // 2, 4), lambda i: (0, 0)),
        pl.BlockSpec((1, 4), lambda i: (0, 0)),
    ],
    out_specs=pl.BlockSpec((G, 4), lambda i: (0, 0)),
    out_shape=jax.ShapeDtypeStruct((G, 4), jnp.float32),
    scratch_shapes=[
        pltpu.VMEM((G, NHID), jnp.float32),
        pltpu.VMEM((G, NHID), jnp.float32),
        pltpu.VMEM((G, NHID), jnp.float32),
        pltpu.VMEM((G, NHID), jnp.float32),
    ],
)


def kernel(x, edge_index, pin_feature, fake_pos, batch, macro_index, meta_feature,
           W1, Wp, b1, Wm, bm, M1W, M1b, M2W, M2b, M3W, M3b):
    f32 = jnp.float32
    ei = edge_index.astype(jnp.int32)
    node = ei[0]
    he = ei[1]
    batch2 = batch.astype(jnp.int32).reshape(N, 1)
    macro32 = macro_index.astype(jnp.int32)

    i32 = jnp.int32

    def _pad_chunks(v, padval):
        return jnp.concatenate(
            [v.reshape(NW, PPW), jnp.full((NW, PAD1), padval, i32)], axis=1
        ).reshape(NW, NCHK1, CHUNK)

    node_g = _pad_chunks(node, 0)
    node_s = _pad_chunks(node, N)
    he_g = _pad_chunks(he, 0)
    he_s = _pad_chunks(he, N)

    pin4 = jnp.concatenate(
        [pin_feature.reshape(NW, PPW, 4).transpose(0, 2, 1),
         jnp.zeros((NW, 4, PAD1), f32)], axis=2).reshape(NW, 4, NCHK1, CHUNK)

    ones128 = jnp.ones((CHUNK,), f32)
    z1 = jnp.zeros((RPTW,), f32)
    z128 = jnp.zeros((RPT, NHID), f32)

    sc_counts, sc_spmm = _sc_kernels()
    (f00, f10, f20, f30, ct0, f01, f11, f21, f31, ct1,
     nd0, nd1, mc0, mc1) = sc_counts(node_s, he_s, pin4, ones128, macro32, z1)
    php = jnp.stack([
        jnp.stack([f00[:N], f10[:N], f20[:N], f30[:N], ct0[:N]], axis=-1),
        jnp.stack([f01[:N], f11[:N], f21[:N], f31[:N], ct1[:N]], axis=-1)])
    ncp = jnp.stack([nd0[:N], nd1[:N]])[:, :, None]
    mcp = jnp.stack([mc0[:N], mc1[:N]])[:, :, None]

    xin = jnp.concatenate([x, fake_pos, jnp.zeros((N, 1), f32)], axis=1)
    xw = _tc_xw(xin, W1, W1[NHID - 1:NHID, :], mcp)

    me_part = sc_spmm(xw, node_g, he_s, z128)

    wp5 = jnp.concatenate([Wp, jnp.zeros((1, NHID), f32)], axis=0)
    m_e = _tc_me(me_part, php, wp5)

    out_part = sc_spmm(m_e, he_g, node_s, z128)

    return _tc_poolhead(
        out_part, ncp, mcp, b1.reshape(1, NHID), batch2, meta_feature,
        Wm, bm.reshape(1, NHID), M1W, M1b.reshape(1, NHID),
        M2W, M2b.reshape(1, NHID // 2), M3W, M3b.reshape(1, 4))
